# baseline (speedup 1.0000x reference)
import functools

import jax
import jax.numpy as jnp
from jax import lax
from jax.experimental import pallas as pl
from jax.experimental.pallas import tpu as pltpu
from jax.experimental.pallas import tpu_sc as plsc

K_NN = 16
I32MAX = 2**31 - 1
LANES = 16


def _dist_body(src_ref, dst_ref, dist_ref):
    src = src_ref[...]
    dst = dst_ref[...]
    s2 = jnp.sum(src * src, axis=-1, keepdims=True)
    d2 = jnp.sum(dst * dst, axis=-1)[None, :]
    ab = lax.dot_general(src, dst, (((1,), (1,)), ((), ())),
                         preferred_element_type=jnp.float32)
    dist2 = jnp.maximum(s2 + d2 - 2.0 * ab, 0.0)
    safe = jnp.where(dist2 > 0, dist2, 1.0)
    dist_ref[...] = jnp.where(dist2 > 0, jnp.sqrt(safe), 0.0)


def _tc_dist(src, dst):
    q, d = src.shape
    n, _ = dst.shape
    qb = min(128, q)
    return pl.pallas_call(
        _dist_body,
        grid=(q // qb,),
        in_specs=[
            pl.BlockSpec((qb, d), lambda i: (i, 0)),
            pl.BlockSpec((n, d), lambda i: (0, 0)),
        ],
        out_specs=pl.BlockSpec((qb, n), lambda i: (i, 0)),
        out_shape=jax.ShapeDtypeStruct((q, n), jnp.float32),
        compiler_params=pltpu.CompilerParams(
            dimension_semantics=("arbitrary",),
        ),
    )(src, dst)


def _rot(v, lane, sh):
    idx = (lane + sh) & (LANES - 1)
    return v.at[idx].get(mode="promise_in_bounds")


def _tree_max(v, lane):
    for sh in (8, 4, 2, 1):
        v = jnp.maximum(v, _rot(v, lane, sh))
    return v


def _sc_topk_build(q, n):
    nblk = n // LANES
    nw = 32
    rpw = q // nw
    capl = n // LANES
    mesh = plsc.VectorSubcoreMesh(core_axis_name="c", subcore_axis_name="s")

    @functools.partial(
        pl.kernel,
        mesh=mesh,
        compiler_params=pltpu.CompilerParams(needs_layout_passes=False),
        out_type=[
            jax.ShapeDtypeStruct((q, K_NN), jnp.float32),
            jax.ShapeDtypeStruct((q, K_NN), jnp.int32),
        ],
        scratch_types=[
            pltpu.VMEM((n,), jnp.float32),
            pltpu.VMEM((LANES * capl,), jnp.float32),
            pltpu.VMEM((LANES * capl,), jnp.int32),
            pltpu.VMEM((rpw, K_NN), jnp.float32),
            pltpu.VMEM((rpw, K_NN), jnp.int32),
        ],
    )
    def sc_topk(dist_hbm, vals_hbm, idx_hbm, row_v, cand_v, cand_i,
                out_v, out_i):
        wid = lax.axis_index("s") * 2 + lax.axis_index("c")
        row0 = wid * rpw
        lane = lax.broadcasted_iota(jnp.int32, (LANES,), 0)
        base = lane * capl
        inf_vec = jnp.full((LANES,), jnp.inf, jnp.float32)
        imax_vec = jnp.full((LANES,), I32MAX, jnp.int32)

        def per_row(r_local, carry):
            pltpu.sync_copy(dist_hbm.at[row0 + r_local], row_v)

            def pa(j, mn):
                off = pl.multiple_of(j * LANES, LANES)
                return jnp.minimum(mn, row_v[pl.ds(off, LANES)])
            mn = lax.fori_loop(0, nblk, pa, inf_vec)
            t = _tree_max(mn, lane)

            def pb(j, cnt_l):
                off = pl.multiple_of(j * LANES, LANES)
                v = row_v[pl.ds(off, LANES)]
                mask = v <= t
                plsc.store_scatter(cand_v, [base + cnt_l], v, mask=mask)
                plsc.store_scatter(cand_i, [base + cnt_l], lane + j * LANES,
                                   mask=mask)
                return cnt_l + jnp.where(mask, jnp.int32(1), jnp.int32(0))
            cnt_l = lax.fori_loop(0, nblk, pb, jnp.zeros((LANES,), jnp.int32))
            maxc = _tree_max(cnt_l, lane)
            nv = maxc[0]

            rv, ri = inf_vec, imax_vec
            pv = jnp.full((LANES,), -jnp.inf, jnp.float32)
            pi = jnp.full((LANES,), -1, jnp.int32)
            for tt in range(K_NN):
                def pc_body(j, bvbi, pv=pv, pi=pi):
                    bv, bi = bvbi
                    gi = base + j
                    v = plsc.load_gather(cand_v, [gi])
                    i = plsc.load_gather(cand_i, [gi])
                    gt = (v > pv) | ((v == pv) & (i > pi))
                    ok = (j < cnt_l) & gt
                    vv = jnp.where(ok, v, jnp.inf)
                    ii = jnp.where(ok, i, I32MAX)
                    lt = (vv < bv) | ((vv == bv) & (ii < bi))
                    return jnp.where(lt, vv, bv), jnp.where(lt, ii, bi)
                bv, bi = lax.fori_loop(0, nv, pc_body, (inf_vec, imax_vec))
                for sh in (8, 4, 2, 1):
                    ov = _rot(bv, lane, sh)
                    oi = _rot(bi, lane, sh)
                    pick = (ov < bv) | ((ov == bv) & (oi < bi))
                    bv = jnp.where(pick, ov, bv)
                    bi = jnp.where(pick, oi, bi)
                rv = jnp.where(lane == tt, bv, rv)
                ri = jnp.where(lane == tt, bi, ri)
                pv, pi = bv, bi
            out_v[r_local] = rv
            out_i[r_local] = ri
            return carry

        lax.fori_loop(0, rpw, per_row, jnp.int32(0))
        pltpu.sync_copy(out_v, vals_hbm.at[pl.ds(row0, rpw)])
        pltpu.sync_copy(out_i, idx_hbm.at[pl.ds(row0, rpw)])

    return sc_topk


def kernel(src, dst):
    q, _ = src.shape
    n, _ = dst.shape
    dist = _tc_dist(src, dst)
    vals, idx = _sc_topk_build(q, n)(dist)
    return vals, idx


# speedup vs baseline: 1.8984x; 1.8984x over previous
import functools

import jax
import jax.numpy as jnp
from jax import lax
from jax.experimental import pallas as pl
from jax.experimental.pallas import tpu as pltpu
from jax.experimental.pallas import tpu_sc as plsc

K_NN = 16
I32MAX = 2**31 - 1
LANES = 16
CHUNK = 128


def _dist_body(src_ref, dst_ref, dist_ref, cmins_ref):
    src = src_ref[...]
    dst = dst_ref[...]
    qb = src.shape[0]
    nb = dst.shape[0]
    nchb = nb // CHUNK
    s2 = jnp.sum(src * src, axis=-1, keepdims=True)
    d2 = jnp.sum(dst * dst, axis=-1)[None, :]
    ab = lax.dot_general(src, dst, (((1,), (1,)), ((), ())),
                         preferred_element_type=jnp.float32)
    dist2 = jnp.maximum(s2 + d2 - 2.0 * ab, 0.0)
    safe = jnp.where(dist2 > 0, dist2, 1.0)
    dist = jnp.where(dist2 > 0, jnp.sqrt(safe), 0.0)
    dist_ref[...] = dist
    cmins_ref[0] = jnp.min(dist.reshape(qb, nchb, CHUNK), axis=2)


def _thresh_body(cmins_ref, thresh_ref):
    w = cmins_ref[...]
    qb = w.shape[1]
    for _ in range(K_NN - 1):
        m = jnp.min(w, axis=(0, 2), keepdims=True)
        w = jnp.where(w == m, jnp.inf, w)
    t16 = jnp.min(w, axis=(0, 2))
    thresh_ref[...] = jnp.broadcast_to(t16.reshape(qb, 1), (qb, LANES))


def _tc_dist(src, dst):
    q, d = src.shape
    n, _ = dst.shape
    qb = min(128, q)
    nb = min(4096, n)
    nsteps = n // nb
    nch = n // CHUNK
    nchb = nb // CHUNK
    dist, cmins = pl.pallas_call(
        _dist_body,
        grid=(q // qb, nsteps),
        in_specs=[
            pl.BlockSpec((qb, d), lambda i, j: (i, 0)),
            pl.BlockSpec((nb, d), lambda i, j: (j, 0)),
        ],
        out_specs=[
            pl.BlockSpec((qb, nb), lambda i, j: (i, j)),
            pl.BlockSpec((1, qb, nchb), lambda i, j: (j, i, 0)),
        ],
        out_shape=[
            jax.ShapeDtypeStruct((q, n), jnp.float32),
            jax.ShapeDtypeStruct((nsteps, q, nchb), jnp.float32),
        ],
        compiler_params=pltpu.CompilerParams(
            dimension_semantics=("parallel", "arbitrary"),
        ),
    )(src, dst)
    thresh = pl.pallas_call(
        _thresh_body,
        grid=(q // 512,),
        in_specs=[pl.BlockSpec((nsteps, 512, nchb), lambda i: (0, i, 0))],
        out_specs=pl.BlockSpec((512, LANES), lambda i: (i, 0)),
        out_shape=jax.ShapeDtypeStruct((q, LANES), jnp.float32),
        compiler_params=pltpu.CompilerParams(
            dimension_semantics=("arbitrary",),
        ),
    )(cmins)
    return dist, cmins, thresh


def _gat(v, idx):
    return v.at[idx].get(mode="promise_in_bounds")


def _tree_max(v, lane):
    for sh in (8, 4, 2, 1):
        v = jnp.maximum(v, _gat(v, (lane + sh) & (LANES - 1)))
    return v


def _sc_topk_build(q, n):
    nch = n // CHUNK
    nsteps = n // min(4096, n)
    nchb = nch // nsteps
    nw = 32
    rpw = q // nw
    capl = n // LANES
    mesh = plsc.VectorSubcoreMesh(core_axis_name="c", subcore_axis_name="s")

    @functools.partial(
        pl.kernel,
        mesh=mesh,
        compiler_params=pltpu.CompilerParams(needs_layout_passes=False),
        out_type=[
            jax.ShapeDtypeStruct((q, K_NN), jnp.float32),
            jax.ShapeDtypeStruct((q, K_NN), jnp.int32),
        ],
        scratch_types=[
            pltpu.VMEM((n,), jnp.float32),
            pltpu.VMEM((n,), jnp.float32),
            pltpu.VMEM((nch,), jnp.float32),
            pltpu.VMEM((nch,), jnp.float32),
            pltpu.VMEM((rpw * LANES,), jnp.float32),
            pltpu.VMEM((LANES * capl,), jnp.float32),
            pltpu.VMEM((LANES * capl,), jnp.int32),
            pltpu.VMEM((rpw, K_NN), jnp.float32),
            pltpu.VMEM((rpw, K_NN), jnp.int32),
            pltpu.SemaphoreType.DMA,
            pltpu.SemaphoreType.DMA,
        ],
    )
    def sc_topk(dist_hbm, cmins_hbm, thresh_hbm, vals_hbm, idx_hbm,
                row_a, row_b, cm_a, cm_b, th_v, cand_v, cand_i,
                out_v, out_i, sem_a, sem_b):
        wid = lax.axis_index("s") * 2 + lax.axis_index("c")
        row0 = wid * rpw
        lane = lax.broadcasted_iota(jnp.int32, (LANES,), 0)
        base = lane * capl
        inf_vec = jnp.full((LANES,), jnp.inf, jnp.float32)
        imax_vec = jnp.full((LANES,), I32MAX, jnp.int32)
        zero_i = jnp.zeros((LANES,), jnp.int32)

        pltpu.sync_copy(thresh_hbm.at[pl.ds(row0 * LANES, rpw * LANES)], th_v)

        def process(row_v, cm_v, r_local):
            toff = pl.multiple_of(r_local * LANES, LANES)
            th_vec = th_v[pl.ds(toff, LANES)]
            th = th_vec[0]

            def pb_chunk(c, cnt_l):
                cm = plsc.load_gather(cm_v, [zero_i + c])[0]

                def hit(cl):
                    for u in range(CHUNK // LANES):
                        off = pl.multiple_of(c * CHUNK + u * LANES, LANES)
                        v = row_v[pl.ds(off, LANES)]
                        mask = v <= th_vec
                        plsc.store_scatter(cand_v, [base + cl], v, mask=mask)
                        plsc.store_scatter(cand_i, [base + cl],
                                           lane + (c * CHUNK + u * LANES),
                                           mask=mask)
                        cl = cl + jnp.where(mask, jnp.int32(1), jnp.int32(0))
                    return cl

                return lax.cond(cm <= th, hit, lambda cl: cl, cnt_l)

            cnt_l = lax.fori_loop(0, nch, pb_chunk, zero_i)
            maxc = _tree_max(cnt_l, lane)[0]

            rv, ri = inf_vec, imax_vec
            pv = jnp.full((LANES,), -jnp.inf, jnp.float32)
            pi = jnp.full((LANES,), -1, jnp.int32)
            for tt in range(K_NN):
                def ext(j, bvbi, pv=pv, pi=pi):
                    bv, bi = bvbi
                    v = plsc.load_gather(cand_v, [base + j])
                    i = plsc.load_gather(cand_i, [base + j])
                    gt = (v > pv) | ((v == pv) & (i > pi))
                    ok = (j < cnt_l) & gt
                    vv = jnp.where(ok, v, jnp.inf)
                    ii = jnp.where(ok, i, I32MAX)
                    lt = (vv < bv) | ((vv == bv) & (ii < bi))
                    return jnp.where(lt, vv, bv), jnp.where(lt, ii, bi)
                bv, bi = lax.fori_loop(0, maxc, ext, (inf_vec, imax_vec))
                for sh in (8, 4, 2, 1):
                    ov = _gat(bv, (lane + sh) & (LANES - 1))
                    oi = _gat(bi, (lane + sh) & (LANES - 1))
                    pick = (ov < bv) | ((ov == bv) & (oi < bi))
                    bv = jnp.where(pick, ov, bv)
                    bi = jnp.where(pick, oi, bi)
                rv = jnp.where(lane == tt, bv, rv)
                ri = jnp.where(lane == tt, bi, ri)
                pv, pi = bv, bi
            out_v[r_local] = rv
            out_i[r_local] = ri

        def fetch(r, row_buf, cm_buf, sem):
            rg = row0 + r
            pltpu.async_copy(dist_hbm.at[rg], row_buf, sem)
            for j in range(nsteps):
                pltpu.async_copy(
                    cmins_hbm.at[pl.ds(j * q * nchb + rg * nchb, nchb)],
                    cm_buf.at[pl.ds(j * nchb, nchb)], sem)

        def drain(row_buf, cm_buf, sem):
            pltpu.make_async_copy(dist_hbm.at[0], row_buf, sem).wait()
            pltpu.make_async_copy(
                cmins_hbm.at[pl.ds(0, nch)], cm_buf, sem).wait()

        fetch(0, row_a, cm_a, sem_a)

        def pair(rp, carry):
            r = 2 * rp
            drain(row_a, cm_a, sem_a)
            fetch(r + 1, row_b, cm_b, sem_b)
            process(row_a, cm_a, r)
            drain(row_b, cm_b, sem_b)
            fetch(jnp.minimum(r + 2, rpw - 1), row_a, cm_a, sem_a)
            process(row_b, cm_b, r + 1)
            return carry

        lax.fori_loop(0, rpw // 2, pair, jnp.int32(0))
        drain(row_a, cm_a, sem_a)

        pltpu.sync_copy(out_v, vals_hbm.at[pl.ds(row0, rpw)])
        pltpu.sync_copy(out_i, idx_hbm.at[pl.ds(row0, rpw)])

    return sc_topk


def kernel(src, dst):
    q, _ = src.shape
    n, _ = dst.shape
    dist, cmins, thresh = _tc_dist(src, dst)
    vals, idx = _sc_topk_build(q, n)(
        dist, cmins.reshape(-1), thresh.reshape(-1))
    return vals, idx


# speedup vs baseline: 2.4171x; 1.2732x over previous
import functools

import jax
import jax.numpy as jnp
from jax import lax
from jax.experimental import pallas as pl
from jax.experimental.pallas import tpu as pltpu
from jax.experimental.pallas import tpu_sc as plsc

K_NN = 16
I32MAX = 2**31 - 1
LANES = 16
CHUNK = 128


def _dist_body(src_ref, dst_ref, dist_ref, cmins_ref):
    src = src_ref[...]
    dst = dst_ref[...]
    qb = src.shape[0]
    nb = dst.shape[0]
    nchb = nb // CHUNK
    s2 = jnp.sum(src * src, axis=-1, keepdims=True)
    d2 = jnp.sum(dst * dst, axis=-1)[None, :]
    ab = lax.dot_general(src, dst, (((1,), (1,)), ((), ())),
                         preferred_element_type=jnp.float32)
    dist2 = jnp.maximum(s2 + d2 - 2.0 * ab, 0.0)
    safe = jnp.where(dist2 > 0, dist2, 1.0)
    dist = jnp.where(dist2 > 0, jnp.sqrt(safe), 0.0)
    dist_ref[...] = dist
    cmins_ref[0] = jnp.min(dist.reshape(qb, nchb, CHUNK), axis=2)


def _thresh_body(cmins_ref, thresh_ref):
    w = cmins_ref[...]
    qb = w.shape[1]
    for _ in range(K_NN - 1):
        m = jnp.min(w, axis=(0, 2), keepdims=True)
        w = jnp.where(w == m, jnp.inf, w)
    t16 = jnp.min(w, axis=(0, 2))
    thresh_ref[...] = jnp.broadcast_to(t16.reshape(qb, 1), (qb, LANES))


def _tc_dist(src, dst):
    q, d = src.shape
    n, _ = dst.shape
    qb = min(128, q)
    nb = min(4096, n)
    nsteps = n // nb
    nch = n // CHUNK
    nchb = nb // CHUNK
    dist, cmins = pl.pallas_call(
        _dist_body,
        grid=(q // qb, nsteps),
        in_specs=[
            pl.BlockSpec((qb, d), lambda i, j: (i, 0)),
            pl.BlockSpec((nb, d), lambda i, j: (j, 0)),
        ],
        out_specs=[
            pl.BlockSpec((qb, nb), lambda i, j: (i, j)),
            pl.BlockSpec((1, qb, nchb), lambda i, j: (j, i, 0)),
        ],
        out_shape=[
            jax.ShapeDtypeStruct((q, n), jnp.float32),
            jax.ShapeDtypeStruct((nsteps, q, nchb), jnp.float32),
        ],
        compiler_params=pltpu.CompilerParams(
            dimension_semantics=("parallel", "arbitrary"),
        ),
    )(src, dst)
    thresh = pl.pallas_call(
        _thresh_body,
        grid=(q // 512,),
        in_specs=[pl.BlockSpec((nsteps, 512, nchb), lambda i: (0, i, 0))],
        out_specs=pl.BlockSpec((512, LANES), lambda i: (i, 0)),
        out_shape=jax.ShapeDtypeStruct((q, LANES), jnp.float32),
        compiler_params=pltpu.CompilerParams(
            dimension_semantics=("arbitrary",),
        ),
    )(cmins)
    return dist, cmins, thresh


def _gat(v, idx):
    return v.at[idx].get(mode="promise_in_bounds")


def _tree_max(v, lane):
    for sh in (8, 4, 2, 1):
        v = jnp.maximum(v, _gat(v, (lane + sh) & (LANES - 1)))
    return v


def _sc_topk_build(q, n):
    nch = n // CHUNK
    nsteps = n // min(4096, n)
    nchb = nch // nsteps
    nw = 32
    rpw = q // nw
    capl = n // LANES
    mesh = plsc.VectorSubcoreMesh(core_axis_name="c", subcore_axis_name="s")

    @functools.partial(
        pl.kernel,
        mesh=mesh,
        compiler_params=pltpu.CompilerParams(needs_layout_passes=False),
        out_type=[
            jax.ShapeDtypeStruct((q, K_NN), jnp.float32),
            jax.ShapeDtypeStruct((q, K_NN), jnp.int32),
        ],
        scratch_types=[
            pltpu.VMEM((n,), jnp.float32),
            pltpu.VMEM((n,), jnp.float32),
            pltpu.VMEM((nch,), jnp.float32),
            pltpu.VMEM((nch,), jnp.float32),
            pltpu.VMEM((rpw * LANES,), jnp.float32),
            pltpu.VMEM((LANES * capl,), jnp.float32),
            pltpu.VMEM((LANES * capl,), jnp.int32),
            pltpu.VMEM((nch,), jnp.int32),
            pltpu.VMEM((nch + LANES,), jnp.int32),
            pltpu.VMEM((rpw, K_NN), jnp.float32),
            pltpu.VMEM((rpw, K_NN), jnp.int32),
            pltpu.SemaphoreType.DMA,
            pltpu.SemaphoreType.DMA,
        ],
    )
    def sc_topk(dist_hbm, cmins_hbm, thresh_hbm, vals_hbm, idx_hbm,
                row_a, row_b, cm_a, cm_b, th_v, cand_v, cand_i, hl_v, hlc_v,
                out_v, out_i, sem_a, sem_b):
        wid = lax.axis_index("s") * 2 + lax.axis_index("c")
        row0 = wid * rpw
        lane = lax.broadcasted_iota(jnp.int32, (LANES,), 0)
        base = lane * capl
        inf_vec = jnp.full((LANES,), jnp.inf, jnp.float32)
        imax_vec = jnp.full((LANES,), I32MAX, jnp.int32)
        zero_i = jnp.zeros((LANES,), jnp.int32)

        pltpu.sync_copy(thresh_hbm.at[pl.ds(row0 * LANES, rpw * LANES)], th_v)

        def process(row_v, cm_v, r_local):
            toff = pl.multiple_of(r_local * LANES, LANES)
            th_vec = th_v[pl.ds(toff, LANES)]
            th = th_vec[0]

            caph = nch // LANES

            def hb(u, hcnt):
                cmv = cm_v[pl.ds(pl.multiple_of(u * LANES, LANES), LANES)]
                mask = cmv <= th_vec
                plsc.store_scatter(hl_v, [lane * caph + hcnt],
                                   lane + u * LANES, mask=mask)
                return hcnt + jnp.where(mask, jnp.int32(1), jnp.int32(0))
            hcnt = lax.fori_loop(0, caph, hb, zero_i)

            s = hcnt
            for sh in (1, 2, 4, 8):
                g = _gat(s, (lane - sh) & (LANES - 1))
                s = s + jnp.where(lane >= sh, g, 0)
            excl = s - hcnt
            nhit = s[LANES - 1]
            maxh = _tree_max(hcnt, lane)[0]

            def hcomp(k, carry):
                m = k < hcnt
                ids = plsc.load_gather(hl_v, [lane * caph + k])
                plsc.store_scatter(hlc_v, [excl + k], ids, mask=m)
                return carry
            lax.fori_loop(0, maxh, hcomp, jnp.int32(0))

            def pb2(it, cnt_l):
                hc = plsc.load_gather(hlc_v, [zero_i + it])[0]
                cb = pl.multiple_of(hc * CHUNK, CHUNK)
                for u in range(CHUNK // LANES):
                    v = row_v[pl.ds(cb + u * LANES, LANES)]
                    mask = v <= th_vec
                    plsc.store_scatter(cand_v, [base + cnt_l], v, mask=mask)
                    plsc.store_scatter(cand_i, [base + cnt_l],
                                       lane + (hc * CHUNK + u * LANES),
                                       mask=mask)
                    cnt_l = cnt_l + jnp.where(mask, jnp.int32(1),
                                              jnp.int32(0))
                return cnt_l

            cnt_l = lax.fori_loop(0, nhit, pb2, zero_i)
            maxc = _tree_max(cnt_l, lane)[0]

            rv, ri = inf_vec, imax_vec
            pv = jnp.full((LANES,), -jnp.inf, jnp.float32)
            pi = jnp.full((LANES,), -1, jnp.int32)
            for tt in range(K_NN):
                def ext(j, bvbi, pv=pv, pi=pi):
                    bv, bi = bvbi
                    v = plsc.load_gather(cand_v, [base + j])
                    i = plsc.load_gather(cand_i, [base + j])
                    gt = (v > pv) | ((v == pv) & (i > pi))
                    ok = (j < cnt_l) & gt
                    vv = jnp.where(ok, v, jnp.inf)
                    ii = jnp.where(ok, i, I32MAX)
                    lt = (vv < bv) | ((vv == bv) & (ii < bi))
                    return jnp.where(lt, vv, bv), jnp.where(lt, ii, bi)
                bv, bi = lax.fori_loop(0, maxc, ext, (inf_vec, imax_vec))
                for sh in (8, 4, 2, 1):
                    ov = _gat(bv, (lane + sh) & (LANES - 1))
                    oi = _gat(bi, (lane + sh) & (LANES - 1))
                    pick = (ov < bv) | ((ov == bv) & (oi < bi))
                    bv = jnp.where(pick, ov, bv)
                    bi = jnp.where(pick, oi, bi)
                rv = jnp.where(lane == tt, bv, rv)
                ri = jnp.where(lane == tt, bi, ri)
                pv, pi = bv, bi
            out_v[r_local] = rv
            out_i[r_local] = ri

        def fetch(r, row_buf, cm_buf, sem):
            rg = row0 + r
            pltpu.async_copy(dist_hbm.at[rg], row_buf, sem)
            for j in range(nsteps):
                pltpu.async_copy(
                    cmins_hbm.at[pl.ds(j * q * nchb + rg * nchb, nchb)],
                    cm_buf.at[pl.ds(j * nchb, nchb)], sem)

        def drain(row_buf, cm_buf, sem):
            pltpu.make_async_copy(dist_hbm.at[0], row_buf, sem).wait()
            pltpu.make_async_copy(
                cmins_hbm.at[pl.ds(0, nch)], cm_buf, sem).wait()

        fetch(0, row_a, cm_a, sem_a)

        def pair(rp, carry):
            r = 2 * rp
            drain(row_a, cm_a, sem_a)
            fetch(r + 1, row_b, cm_b, sem_b)
            process(row_a, cm_a, r)
            drain(row_b, cm_b, sem_b)
            fetch(jnp.minimum(r + 2, rpw - 1), row_a, cm_a, sem_a)
            process(row_b, cm_b, r + 1)
            return carry

        lax.fori_loop(0, rpw // 2, pair, jnp.int32(0))
        drain(row_a, cm_a, sem_a)

        pltpu.sync_copy(out_v, vals_hbm.at[pl.ds(row0, rpw)])
        pltpu.sync_copy(out_i, idx_hbm.at[pl.ds(row0, rpw)])

    return sc_topk


def kernel(src, dst):
    q, _ = src.shape
    n, _ = dst.shape
    dist, cmins, thresh = _tc_dist(src, dst)
    vals, idx = _sc_topk_build(q, n)(
        dist, cmins.reshape(-1), thresh.reshape(-1))
    return vals, idx


# speedup vs baseline: 3.3614x; 1.3907x over previous
import functools

import jax
import jax.numpy as jnp
from jax import lax
from jax.experimental import pallas as pl
from jax.experimental.pallas import tpu as pltpu
from jax.experimental.pallas import tpu_sc as plsc

K_NN = 16
I32MAX = 2**31 - 1
LANES = 16
CHUNK = 128


def _dist_body(src_ref, dst_ref, dist_ref, cmins_ref):
    src = src_ref[...]
    dst = dst_ref[...]
    qb = src.shape[0]
    nb = dst.shape[0]
    nchb = nb // CHUNK
    s2 = jnp.sum(src * src, axis=-1, keepdims=True)
    d2 = jnp.sum(dst * dst, axis=-1)[None, :]
    ab = lax.dot_general(src, dst, (((1,), (1,)), ((), ())),
                         preferred_element_type=jnp.float32)
    dist2 = jnp.maximum(s2 + d2 - 2.0 * ab, 0.0)
    safe = jnp.where(dist2 > 0, dist2, 1.0)
    dist = jnp.where(dist2 > 0, jnp.sqrt(safe), 0.0)
    dist_ref[...] = dist
    cmins_ref[0] = jnp.min(dist.reshape(qb, nchb, CHUNK), axis=2)


def _thresh_body(cmins_ref, thresh_ref):
    w = cmins_ref[...]
    qb = w.shape[1]
    for _ in range(K_NN - 1):
        m = jnp.min(w, axis=(0, 2), keepdims=True)
        w = jnp.where(w == m, jnp.inf, w)
    t16 = jnp.min(w, axis=(0, 2))
    thresh_ref[...] = jnp.broadcast_to(t16.reshape(qb, 1), (qb, LANES))


def _tc_dist(src, dst):
    q, d = src.shape
    n, _ = dst.shape
    qb = min(128, q)
    nb = min(4096, n)
    nsteps = n // nb
    nch = n // CHUNK
    nchb = nb // CHUNK
    dist, cmins = pl.pallas_call(
        _dist_body,
        grid=(q // qb, nsteps),
        in_specs=[
            pl.BlockSpec((qb, d), lambda i, j: (i, 0)),
            pl.BlockSpec((nb, d), lambda i, j: (j, 0)),
        ],
        out_specs=[
            pl.BlockSpec((qb, nb), lambda i, j: (i, j)),
            pl.BlockSpec((1, qb, nchb), lambda i, j: (j, i, 0)),
        ],
        out_shape=[
            jax.ShapeDtypeStruct((q, n), jnp.float32),
            jax.ShapeDtypeStruct((nsteps, q, nchb), jnp.float32),
        ],
        compiler_params=pltpu.CompilerParams(
            dimension_semantics=("parallel", "arbitrary"),
        ),
    )(src, dst)
    thresh = pl.pallas_call(
        _thresh_body,
        grid=(q // 512,),
        in_specs=[pl.BlockSpec((nsteps, 512, nchb), lambda i: (0, i, 0))],
        out_specs=pl.BlockSpec((512, LANES), lambda i: (i, 0)),
        out_shape=jax.ShapeDtypeStruct((q, LANES), jnp.float32),
        compiler_params=pltpu.CompilerParams(
            dimension_semantics=("arbitrary",),
        ),
    )(cmins)
    return dist, cmins, thresh


def _gat(v, idx):
    return v.at[idx].get(mode="promise_in_bounds")


def _tree_max(v, lane):
    for sh in (8, 4, 2, 1):
        v = jnp.maximum(v, _gat(v, (lane + sh) & (LANES - 1)))
    return v


def _sc_topk_build(q, n):
    nch = n // CHUNK
    nsteps = n // min(4096, n)
    nchb = nch // nsteps
    nw = 32
    rpw = q // nw
    capl = n // LANES
    mesh = plsc.VectorSubcoreMesh(core_axis_name="c", subcore_axis_name="s")

    @functools.partial(
        pl.kernel,
        mesh=mesh,
        compiler_params=pltpu.CompilerParams(needs_layout_passes=False),
        out_type=[
            jax.ShapeDtypeStruct((q, K_NN), jnp.float32),
            jax.ShapeDtypeStruct((q, K_NN), jnp.int32),
        ],
        scratch_types=[
            pltpu.VMEM((n,), jnp.float32),
            pltpu.VMEM((n,), jnp.float32),
            pltpu.VMEM((nch,), jnp.float32),
            pltpu.VMEM((nch,), jnp.float32),
            pltpu.VMEM((rpw * LANES,), jnp.float32),
            pltpu.VMEM((LANES * capl,), jnp.float32),
            pltpu.VMEM((LANES * capl,), jnp.int32),
            pltpu.VMEM((nch,), jnp.int32),
            pltpu.VMEM((nch + LANES,), jnp.int32),
            pltpu.VMEM((rpw, K_NN), jnp.float32),
            pltpu.VMEM((rpw, K_NN), jnp.int32),
            pltpu.SemaphoreType.DMA,
            pltpu.SemaphoreType.DMA,
        ],
    )
    def sc_topk(dist_hbm, cmins_hbm, thresh_hbm, vals_hbm, idx_hbm,
                row_a, row_b, cm_a, cm_b, th_v, cand_v, cand_i, hl_v, hlc_v,
                out_v, out_i, sem_a, sem_b):
        wid = lax.axis_index("s") * 2 + lax.axis_index("c")
        row0 = wid * rpw
        lane = lax.broadcasted_iota(jnp.int32, (LANES,), 0)
        base = lane * capl
        inf_vec = jnp.full((LANES,), jnp.inf, jnp.float32)
        imax_vec = jnp.full((LANES,), I32MAX, jnp.int32)
        zero_i = jnp.zeros((LANES,), jnp.int32)

        pltpu.sync_copy(thresh_hbm.at[pl.ds(row0 * LANES, rpw * LANES)], th_v)

        def process(row_v, cm_v, r_local):
            toff = pl.multiple_of(r_local * LANES, LANES)
            th_vec = th_v[pl.ds(toff, LANES)]
            th = th_vec[0]

            caph = nch // LANES

            def hb(u, hcnt):
                cmv = cm_v[pl.ds(pl.multiple_of(u * LANES, LANES), LANES)]
                mask = cmv <= th_vec
                plsc.store_scatter(hl_v, [lane * caph + hcnt],
                                   lane + u * LANES, mask=mask)
                return hcnt + jnp.where(mask, jnp.int32(1), jnp.int32(0))
            hcnt = lax.fori_loop(0, caph, hb, zero_i)

            s = hcnt
            for sh in (1, 2, 4, 8):
                g = _gat(s, (lane - sh) & (LANES - 1))
                s = s + jnp.where(lane >= sh, g, 0)
            excl = s - hcnt
            nhit = s[LANES - 1]
            maxh = _tree_max(hcnt, lane)[0]

            def hcomp(k, carry):
                m = k < hcnt
                ids = plsc.load_gather(hl_v, [lane * caph + k])
                plsc.store_scatter(hlc_v, [excl + k], ids, mask=m)
                return carry
            lax.fori_loop(0, maxh, hcomp, jnp.int32(0))

            def pb2(it, cnt_l):
                hc = plsc.load_gather(hlc_v, [zero_i + it])[0]
                cb = pl.multiple_of(hc * CHUNK, CHUNK)
                for u in range(CHUNK // LANES):
                    v = row_v[pl.ds(cb + u * LANES, LANES)]
                    mask = v <= th_vec
                    plsc.store_scatter(cand_v, [base + cnt_l], v, mask=mask)
                    plsc.store_scatter(cand_i, [base + cnt_l],
                                       lane + (hc * CHUNK + u * LANES),
                                       mask=mask)
                    cnt_l = cnt_l + jnp.where(mask, jnp.int32(1),
                                              jnp.int32(0))
                return cnt_l

            cnt_l = lax.fori_loop(0, nhit, pb2, zero_i)
            maxc = _tree_max(cnt_l, lane)[0]

            rv, ri = inf_vec, imax_vec
            pv = jnp.full((LANES,), -jnp.inf, jnp.float32)
            pi = jnp.full((LANES,), -1, jnp.int32)
            for tt in range(K_NN):
                def ext(j, bvbi, pv=pv, pi=pi):
                    bv, bi = bvbi
                    v = plsc.load_gather(cand_v, [base + j])
                    i = plsc.load_gather(cand_i, [base + j])
                    gt = (v > pv) | ((v == pv) & (i > pi))
                    ok = (j < cnt_l) & gt
                    vv = jnp.where(ok, v, jnp.inf)
                    ii = jnp.where(ok, i, I32MAX)
                    lt = (vv < bv) | ((vv == bv) & (ii < bi))
                    return jnp.where(lt, vv, bv), jnp.where(lt, ii, bi)
                bv, bi = lax.fori_loop(0, maxc, ext, (inf_vec, imax_vec))
                for sh in (8, 4, 2, 1):
                    ov = _gat(bv, (lane + sh) & (LANES - 1))
                    oi = _gat(bi, (lane + sh) & (LANES - 1))
                    pick = (ov < bv) | ((ov == bv) & (oi < bi))
                    bv = jnp.where(pick, ov, bv)
                    bi = jnp.where(pick, oi, bi)
                rv = jnp.where(lane == tt, bv, rv)
                ri = jnp.where(lane == tt, bi, ri)
                pv, pi = bv, bi
            out_v[r_local] = rv
            out_i[r_local] = ri

        def fetch(r, row_buf, cm_buf, sem):
            rg = row0 + r
            pltpu.async_copy(dist_hbm.at[rg], row_buf, sem)
            for j in range(nsteps):
                pltpu.async_copy(
                    cmins_hbm.at[pl.ds(j * q * nchb + rg * nchb, nchb)],
                    cm_buf.at[pl.ds(j * nchb, nchb)], sem)

        def drain(row_buf, cm_buf, sem):
            pltpu.make_async_copy(dist_hbm.at[0], row_buf, sem).wait()
            pltpu.make_async_copy(
                cmins_hbm.at[pl.ds(0, nch)], cm_buf, sem).wait()

        fetch(0, row_a, cm_a, sem_a)

        def pair(rp, carry):
            r = 2 * rp
            drain(row_a, cm_a, sem_a)
            fetch(r + 1, row_b, cm_b, sem_b)
            process(row_a, cm_a, r)
            drain(row_b, cm_b, sem_b)
            fetch(jnp.minimum(r + 2, rpw - 1), row_a, cm_a, sem_a)
            process(row_b, cm_b, r + 1)
            return carry

        lax.fori_loop(0, rpw // 2, pair, jnp.int32(0))
        drain(row_a, cm_a, sem_a)

        pltpu.sync_copy(out_v, vals_hbm.at[pl.ds(row0, rpw)])
        pltpu.sync_copy(out_i, idx_hbm.at[pl.ds(row0, rpw)])

    return sc_topk


def kernel(src, dst):
    q, _ = src.shape
    n, _ = dst.shape
    groups = 4
    qg = q // groups
    sc_topk = _sc_topk_build(qg, n)
    vals_parts, idx_parts = [], []
    for g in range(groups):
        dist, cmins, thresh = _tc_dist(src[g * qg:(g + 1) * qg], dst)
        v, i = sc_topk(dist, cmins.reshape(-1), thresh.reshape(-1))
        vals_parts.append(v)
        idx_parts.append(i)
    return (jnp.concatenate(vals_parts, axis=0),
            jnp.concatenate(idx_parts, axis=0))


# speedup vs baseline: 3.3754x; 1.0042x over previous
import functools

import jax
import jax.numpy as jnp
from jax import lax
from jax.experimental import pallas as pl
from jax.experimental.pallas import tpu as pltpu
from jax.experimental.pallas import tpu_sc as plsc

K_NN = 16
I32MAX = 2**31 - 1
LANES = 16
CHUNK = 128


def _dist_body(src_ref, dst_ref, dist_ref, cmins_ref):
    src = src_ref[...]
    dst = dst_ref[...]
    qb = src.shape[0]
    nb = dst.shape[0]
    nchb = nb // CHUNK
    s2 = jnp.sum(src * src, axis=-1, keepdims=True)
    d2 = jnp.sum(dst * dst, axis=-1)[None, :]
    ab = lax.dot_general(src, dst, (((1,), (1,)), ((), ())),
                         preferred_element_type=jnp.float32)
    dist2 = jnp.maximum(s2 + d2 - 2.0 * ab, 0.0)
    safe = jnp.where(dist2 > 0, dist2, 1.0)
    dist = jnp.where(dist2 > 0, jnp.sqrt(safe), 0.0)
    dist_ref[...] = dist
    cmins_ref[0] = jnp.min(dist.reshape(qb, nchb, CHUNK), axis=2)


def _thresh_body(cmins_ref, thresh_ref):
    w = cmins_ref[...]
    qb = w.shape[1]
    for _ in range(K_NN - 1):
        m = jnp.min(w, axis=(0, 2), keepdims=True)
        w = jnp.where(w == m, jnp.inf, w)
    t16 = jnp.min(w, axis=(0, 2))
    thresh_ref[...] = jnp.broadcast_to(t16.reshape(qb, 1), (qb, LANES))


def _tc_dist(src, dst):
    q, d = src.shape
    n, _ = dst.shape
    qb = min(128, q)
    nb = min(4096, n)
    nsteps = n // nb
    nch = n // CHUNK
    nchb = nb // CHUNK
    dist, cmins = pl.pallas_call(
        _dist_body,
        grid=(q // qb, nsteps),
        in_specs=[
            pl.BlockSpec((qb, d), lambda i, j: (i, 0)),
            pl.BlockSpec((nb, d), lambda i, j: (j, 0)),
        ],
        out_specs=[
            pl.BlockSpec((qb, nb), lambda i, j: (i, j)),
            pl.BlockSpec((1, qb, nchb), lambda i, j: (j, i, 0)),
        ],
        out_shape=[
            jax.ShapeDtypeStruct((q, n), jnp.float32),
            jax.ShapeDtypeStruct((nsteps, q, nchb), jnp.float32),
        ],
        compiler_params=pltpu.CompilerParams(
            dimension_semantics=("parallel", "arbitrary"),
        ),
    )(src, dst)
    thresh = pl.pallas_call(
        _thresh_body,
        grid=(q // 512,),
        in_specs=[pl.BlockSpec((nsteps, 512, nchb), lambda i: (0, i, 0))],
        out_specs=pl.BlockSpec((512, LANES), lambda i: (i, 0)),
        out_shape=jax.ShapeDtypeStruct((q, LANES), jnp.float32),
        compiler_params=pltpu.CompilerParams(
            dimension_semantics=("arbitrary",),
        ),
    )(cmins)
    return dist, cmins, thresh


def _gat(v, idx):
    return v.at[idx].get(mode="promise_in_bounds")


def _tree_max(v, lane):
    for sh in (8, 4, 2, 1):
        v = jnp.maximum(v, _gat(v, (lane + sh) & (LANES - 1)))
    return v


def _sc_topk_build(q, n):
    nch = n // CHUNK
    nsteps = n // min(4096, n)
    nchb = nch // nsteps
    nw = 32
    rpw = q // nw
    capl = n // LANES
    mesh = plsc.VectorSubcoreMesh(core_axis_name="c", subcore_axis_name="s")

    @functools.partial(
        pl.kernel,
        mesh=mesh,
        compiler_params=pltpu.CompilerParams(needs_layout_passes=False),
        out_type=[
            jax.ShapeDtypeStruct((q, K_NN), jnp.float32),
            jax.ShapeDtypeStruct((q, K_NN), jnp.int32),
        ],
        scratch_types=[
            pltpu.VMEM((n,), jnp.float32),
            pltpu.VMEM((n,), jnp.float32),
            pltpu.VMEM((nch,), jnp.float32),
            pltpu.VMEM((nch,), jnp.float32),
            pltpu.VMEM((rpw * LANES,), jnp.float32),
            pltpu.VMEM((LANES * capl,), jnp.float32),
            pltpu.VMEM((LANES * capl,), jnp.int32),
            pltpu.VMEM((nch,), jnp.int32),
            pltpu.VMEM((nch + LANES,), jnp.int32),
            pltpu.VMEM((rpw, K_NN), jnp.float32),
            pltpu.VMEM((rpw, K_NN), jnp.int32),
            pltpu.SemaphoreType.DMA,
            pltpu.SemaphoreType.DMA,
        ],
    )
    def sc_topk(dist_hbm, cmins_hbm, thresh_hbm, vals_hbm, idx_hbm,
                row_a, row_b, cm_a, cm_b, th_v, cand_v, cand_i, hl_v, hlc_v,
                out_v, out_i, sem_a, sem_b):
        wid = lax.axis_index("s") * 2 + lax.axis_index("c")
        row0 = wid * rpw
        lane = lax.broadcasted_iota(jnp.int32, (LANES,), 0)
        base = lane * capl
        inf_vec = jnp.full((LANES,), jnp.inf, jnp.float32)
        imax_vec = jnp.full((LANES,), I32MAX, jnp.int32)
        zero_i = jnp.zeros((LANES,), jnp.int32)

        pltpu.sync_copy(thresh_hbm.at[pl.ds(row0 * LANES, rpw * LANES)], th_v)

        def process(row_v, cm_v, r_local):
            toff = pl.multiple_of(r_local * LANES, LANES)
            th_vec = th_v[pl.ds(toff, LANES)]
            th = th_vec[0]

            caph = nch // LANES

            def hb(u, hcnt):
                cmv = cm_v[pl.ds(pl.multiple_of(u * LANES, LANES), LANES)]
                mask = cmv <= th_vec
                plsc.store_scatter(hl_v, [lane * caph + hcnt],
                                   lane + u * LANES, mask=mask)
                return hcnt + jnp.where(mask, jnp.int32(1), jnp.int32(0))
            hcnt = lax.fori_loop(0, caph, hb, zero_i)

            s = hcnt
            for sh in (1, 2, 4, 8):
                g = _gat(s, (lane - sh) & (LANES - 1))
                s = s + jnp.where(lane >= sh, g, 0)
            excl = s - hcnt
            nhit = s[LANES - 1]
            maxh = _tree_max(hcnt, lane)[0]

            def hcomp(k, carry):
                m = k < hcnt
                ids = plsc.load_gather(hl_v, [lane * caph + k])
                plsc.store_scatter(hlc_v, [excl + k], ids, mask=m)
                return carry
            lax.fori_loop(0, maxh, hcomp, jnp.int32(0))

            def pb2(it, cnt_l):
                hc = plsc.load_gather(hlc_v, [zero_i + it])[0]
                cb = pl.multiple_of(hc * CHUNK, CHUNK)
                for u in range(CHUNK // LANES):
                    v = row_v[pl.ds(cb + u * LANES, LANES)]
                    mask = v <= th_vec
                    plsc.store_scatter(cand_v, [base + cnt_l], v, mask=mask)
                    plsc.store_scatter(cand_i, [base + cnt_l],
                                       lane + (hc * CHUNK + u * LANES),
                                       mask=mask)
                    cnt_l = cnt_l + jnp.where(mask, jnp.int32(1),
                                              jnp.int32(0))
                return cnt_l

            cnt_l = lax.fori_loop(0, nhit, pb2, zero_i)
            maxc = _tree_max(cnt_l, lane)[0]

            rv, ri = inf_vec, imax_vec
            pv = jnp.full((LANES,), -jnp.inf, jnp.float32)
            pi = jnp.full((LANES,), -1, jnp.int32)
            for tt in range(K_NN):
                def ext(j, bvbi, pv=pv, pi=pi):
                    bv, bi = bvbi
                    v = plsc.load_gather(cand_v, [base + j])
                    i = plsc.load_gather(cand_i, [base + j])
                    gt = (v > pv) | ((v == pv) & (i > pi))
                    ok = (j < cnt_l) & gt
                    vv = jnp.where(ok, v, jnp.inf)
                    ii = jnp.where(ok, i, I32MAX)
                    lt = (vv < bv) | ((vv == bv) & (ii < bi))
                    return jnp.where(lt, vv, bv), jnp.where(lt, ii, bi)
                bv, bi = lax.fori_loop(0, maxc, ext, (inf_vec, imax_vec))
                for sh in (8, 4, 2, 1):
                    ov = _gat(bv, (lane + sh) & (LANES - 1))
                    oi = _gat(bi, (lane + sh) & (LANES - 1))
                    pick = (ov < bv) | ((ov == bv) & (oi < bi))
                    bv = jnp.where(pick, ov, bv)
                    bi = jnp.where(pick, oi, bi)
                rv = jnp.where(lane == tt, bv, rv)
                ri = jnp.where(lane == tt, bi, ri)
                pv, pi = bv, bi
            out_v[r_local] = rv
            out_i[r_local] = ri

        def fetch(r, row_buf, cm_buf, sem):
            rg = row0 + r
            pltpu.async_copy(dist_hbm.at[rg], row_buf, sem)
            for j in range(nsteps):
                pltpu.async_copy(
                    cmins_hbm.at[pl.ds(j * q * nchb + rg * nchb, nchb)],
                    cm_buf.at[pl.ds(j * nchb, nchb)], sem)

        def drain(row_buf, cm_buf, sem):
            pltpu.make_async_copy(dist_hbm.at[0], row_buf, sem).wait()
            pltpu.make_async_copy(
                cmins_hbm.at[pl.ds(0, nch)], cm_buf, sem).wait()

        fetch(0, row_a, cm_a, sem_a)

        def pair(rp, carry):
            r = 2 * rp
            drain(row_a, cm_a, sem_a)
            fetch(r + 1, row_b, cm_b, sem_b)
            process(row_a, cm_a, r)
            drain(row_b, cm_b, sem_b)
            fetch(jnp.minimum(r + 2, rpw - 1), row_a, cm_a, sem_a)
            process(row_b, cm_b, r + 1)
            return carry

        lax.fori_loop(0, rpw // 2, pair, jnp.int32(0))
        drain(row_a, cm_a, sem_a)

        pltpu.sync_copy(out_v, vals_hbm.at[pl.ds(row0, rpw)])
        pltpu.sync_copy(out_i, idx_hbm.at[pl.ds(row0, rpw)])

    return sc_topk


def kernel(src, dst):
    q, _ = src.shape
    n, _ = dst.shape
    groups = 8
    qg = q // groups
    sc_topk = _sc_topk_build(qg, n)
    vals_parts, idx_parts = [], []
    for g in range(groups):
        dist, cmins, thresh = _tc_dist(src[g * qg:(g + 1) * qg], dst)
        v, i = sc_topk(dist, cmins.reshape(-1), thresh.reshape(-1))
        vals_parts.append(v)
        idx_parts.append(i)
    return (jnp.concatenate(vals_parts, axis=0),
            jnp.concatenate(idx_parts, axis=0))


# speedup vs baseline: 4.6000x; 1.3628x over previous
import functools

import jax
import jax.numpy as jnp
from jax import lax
from jax.experimental import pallas as pl
from jax.experimental.pallas import tpu as pltpu
from jax.experimental.pallas import tpu_sc as plsc

K_NN = 16
I32MAX = 2**31 - 1
LANES = 16
CHUNK = 128


def _dist_body(src_ref, dst_ref, dist_ref, cmins_ref):
    src = src_ref[...]
    dst = dst_ref[...]
    qb = src.shape[0]
    nb = dst.shape[0]
    nchb = nb // CHUNK
    s2 = jnp.sum(src * src, axis=-1, keepdims=True)
    d2 = jnp.sum(dst * dst, axis=-1)[None, :]
    ab = lax.dot_general(src, dst, (((1,), (1,)), ((), ())),
                         preferred_element_type=jnp.float32)
    dist2 = jnp.maximum(s2 + d2 - 2.0 * ab, 0.0)
    safe = jnp.where(dist2 > 0, dist2, 1.0)
    dist = jnp.where(dist2 > 0, jnp.sqrt(safe), 0.0)
    dist_ref[...] = dist
    cmins_ref[0] = jnp.min(dist.reshape(qb, nchb, CHUNK), axis=2)


def _thresh_body(cmins_ref, thresh_ref):
    w = cmins_ref[...]
    qb = w.shape[1]
    for _ in range(K_NN - 1):
        m = jnp.min(w, axis=(0, 2), keepdims=True)
        w = jnp.where(w == m, jnp.inf, w)
    t16 = jnp.min(w, axis=(0, 2))
    thresh_ref[...] = jnp.broadcast_to(t16.reshape(qb, 1), (qb, LANES))


def _tc_dist(src, dst):
    q, d = src.shape
    n, _ = dst.shape
    qb = min(256, q)
    nb = min(4096, n)
    nsteps = n // nb
    nch = n // CHUNK
    nchb = nb // CHUNK
    dist, cmins = pl.pallas_call(
        _dist_body,
        grid=(q // qb, nsteps),
        in_specs=[
            pl.BlockSpec((qb, d), lambda i, j: (i, 0)),
            pl.BlockSpec((nb, d), lambda i, j: (j, 0)),
        ],
        out_specs=[
            pl.BlockSpec((qb, nb), lambda i, j: (i, j)),
            pl.BlockSpec((1, qb, nchb), lambda i, j: (j, i, 0)),
        ],
        out_shape=[
            jax.ShapeDtypeStruct((q, n), jnp.float32),
            jax.ShapeDtypeStruct((nsteps, q, nchb), jnp.float32),
        ],
        compiler_params=pltpu.CompilerParams(
            dimension_semantics=("parallel", "arbitrary"),
        ),
    )(src, dst)
    thresh = pl.pallas_call(
        _thresh_body,
        grid=(q // 512,),
        in_specs=[pl.BlockSpec((nsteps, 512, nchb), lambda i: (0, i, 0))],
        out_specs=pl.BlockSpec((512, LANES), lambda i: (i, 0)),
        out_shape=jax.ShapeDtypeStruct((q, LANES), jnp.float32),
        compiler_params=pltpu.CompilerParams(
            dimension_semantics=("arbitrary",),
        ),
    )(cmins)
    return dist, cmins, thresh


def _gat(v, idx):
    return v.at[idx].get(mode="promise_in_bounds")


def _tree_max(v, lane):
    for sh in (8, 4, 2, 1):
        v = jnp.maximum(v, _gat(v, (lane + sh) & (LANES - 1)))
    return v


def _sc_topk_build(q, n):
    nch = n // CHUNK
    nsteps = n // min(4096, n)
    nchb = nch // nsteps
    nw = 32
    rpw = q // nw
    capl = n // LANES
    mesh = plsc.VectorSubcoreMesh(core_axis_name="c", subcore_axis_name="s")

    @functools.partial(
        pl.kernel,
        mesh=mesh,
        compiler_params=pltpu.CompilerParams(needs_layout_passes=False),
        out_type=[
            jax.ShapeDtypeStruct((q, K_NN), jnp.float32),
            jax.ShapeDtypeStruct((q, K_NN), jnp.int32),
        ],
        scratch_types=[
            pltpu.VMEM((n,), jnp.float32),
            pltpu.VMEM((n,), jnp.float32),
            pltpu.VMEM((nch,), jnp.float32),
            pltpu.VMEM((nch,), jnp.float32),
            pltpu.VMEM((rpw * LANES,), jnp.float32),
            pltpu.VMEM((LANES * capl,), jnp.float32),
            pltpu.VMEM((LANES * capl,), jnp.int32),
            pltpu.VMEM((nch,), jnp.int32),
            pltpu.VMEM((nch + LANES,), jnp.int32),
            pltpu.VMEM((rpw, K_NN), jnp.float32),
            pltpu.VMEM((rpw, K_NN), jnp.int32),
            pltpu.SemaphoreType.DMA,
            pltpu.SemaphoreType.DMA,
        ],
    )
    def sc_topk(dist_hbm, cmins_hbm, thresh_hbm, vals_hbm, idx_hbm,
                row_a, row_b, cm_a, cm_b, th_v, cand_v, cand_i, hl_v, hlc_v,
                out_v, out_i, sem_a, sem_b):
        wid = lax.axis_index("s") * 2 + lax.axis_index("c")
        row0 = wid * rpw
        lane = lax.broadcasted_iota(jnp.int32, (LANES,), 0)
        base = lane * capl
        inf_vec = jnp.full((LANES,), jnp.inf, jnp.float32)
        imax_vec = jnp.full((LANES,), I32MAX, jnp.int32)
        zero_i = jnp.zeros((LANES,), jnp.int32)

        pltpu.sync_copy(thresh_hbm.at[pl.ds(row0 * LANES, rpw * LANES)], th_v)

        def process(row_v, cm_v, r_local):
            toff = pl.multiple_of(r_local * LANES, LANES)
            th_vec = th_v[pl.ds(toff, LANES)]
            th = th_vec[0]

            caph = nch // LANES

            def hb(u, hcnt):
                cmv = cm_v[pl.ds(pl.multiple_of(u * LANES, LANES), LANES)]
                mask = cmv <= th_vec
                plsc.store_scatter(hl_v, [lane * caph + hcnt],
                                   lane + u * LANES, mask=mask)
                return hcnt + jnp.where(mask, jnp.int32(1), jnp.int32(0))
            hcnt = lax.fori_loop(0, caph, hb, zero_i)

            s = hcnt
            for sh in (1, 2, 4, 8):
                g = _gat(s, (lane - sh) & (LANES - 1))
                s = s + jnp.where(lane >= sh, g, 0)
            excl = s - hcnt
            nhit = s[LANES - 1]
            maxh = _tree_max(hcnt, lane)[0]

            def hcomp(k, carry):
                m = k < hcnt
                ids = plsc.load_gather(hl_v, [lane * caph + k])
                plsc.store_scatter(hlc_v, [excl + k], ids, mask=m)
                return carry
            lax.fori_loop(0, maxh, hcomp, jnp.int32(0))

            def pb2(it, cnt_l):
                hc = plsc.load_gather(hlc_v, [zero_i + it])[0]
                cb = pl.multiple_of(hc * CHUNK, CHUNK)
                for u in range(CHUNK // LANES):
                    v = row_v[pl.ds(cb + u * LANES, LANES)]
                    mask = v <= th_vec
                    plsc.store_scatter(cand_v, [base + cnt_l], v, mask=mask)
                    plsc.store_scatter(cand_i, [base + cnt_l],
                                       lane + (hc * CHUNK + u * LANES),
                                       mask=mask)
                    cnt_l = cnt_l + jnp.where(mask, jnp.int32(1),
                                              jnp.int32(0))
                return cnt_l

            cnt_l = lax.fori_loop(0, nhit, pb2, zero_i)
            maxc = _tree_max(cnt_l, lane)[0]

            def cstep(v, i, k, j):
                p = lane ^ j
                pv_ = _gat(v, p)
                pi_ = _gat(i, p)
                less = (pv_ < v) | ((pv_ == v) & (pi_ < i))
                grtr = (pv_ > v) | ((pv_ == v) & (pi_ > i))
                lo = (lane & j) == 0
                dir_up = (lane & k) == 0
                takemin = lo == dir_up
                sel = jnp.where(takemin, less, grtr)
                return jnp.where(sel, pv_, v), jnp.where(sel, pi_, i)

            def cmerge(j, rvri):
                rv, ri = rvri
                m = j < cnt_l
                v = plsc.load_gather(cand_v, [base + j])
                i = plsc.load_gather(cand_i, [base + j])
                v = jnp.where(m, v, jnp.inf)
                i = jnp.where(m, i, I32MAX)
                for k, jj in ((2, 1), (4, 2), (4, 1), (8, 4), (8, 2),
                              (8, 1), (16, 8), (16, 4), (16, 2), (16, 1)):
                    v, i = cstep(v, i, k, jj)
                wv = _gat(v, (LANES - 1) - lane)
                wi = _gat(i, (LANES - 1) - lane)
                keep = (rv < wv) | ((rv == wv) & (ri < wi))
                rv = jnp.where(keep, rv, wv)
                ri = jnp.where(keep, ri, wi)
                for jj in (8, 4, 2, 1):
                    rv, ri = cstep(rv, ri, 16, jj)
                return rv, ri

            rv, ri = lax.fori_loop(0, maxc, cmerge, (inf_vec, imax_vec))
            out_v[r_local] = rv
            out_i[r_local] = ri

        def fetch(r, row_buf, cm_buf, sem):
            rg = row0 + r
            pltpu.async_copy(dist_hbm.at[rg], row_buf, sem)
            for j in range(nsteps):
                pltpu.async_copy(
                    cmins_hbm.at[pl.ds(j * q * nchb + rg * nchb, nchb)],
                    cm_buf.at[pl.ds(j * nchb, nchb)], sem)

        def drain(row_buf, cm_buf, sem):
            pltpu.make_async_copy(dist_hbm.at[0], row_buf, sem).wait()
            pltpu.make_async_copy(
                cmins_hbm.at[pl.ds(0, nch)], cm_buf, sem).wait()

        fetch(0, row_a, cm_a, sem_a)

        def pair(rp, carry):
            r = 2 * rp
            drain(row_a, cm_a, sem_a)
            fetch(r + 1, row_b, cm_b, sem_b)
            process(row_a, cm_a, r)
            drain(row_b, cm_b, sem_b)
            fetch(jnp.minimum(r + 2, rpw - 1), row_a, cm_a, sem_a)
            process(row_b, cm_b, r + 1)
            return carry

        lax.fori_loop(0, rpw // 2, pair, jnp.int32(0))
        drain(row_a, cm_a, sem_a)

        pltpu.sync_copy(out_v, vals_hbm.at[pl.ds(row0, rpw)])
        pltpu.sync_copy(out_i, idx_hbm.at[pl.ds(row0, rpw)])

    return sc_topk


def kernel(src, dst):
    q, _ = src.shape
    n, _ = dst.shape
    groups = 8
    qg = q // groups
    sc_topk = _sc_topk_build(qg, n)
    vals_parts, idx_parts = [], []
    for g in range(groups):
        dist, cmins, thresh = _tc_dist(src[g * qg:(g + 1) * qg], dst)
        v, i = sc_topk(dist, cmins.reshape(-1), thresh.reshape(-1))
        vals_parts.append(v)
        idx_parts.append(i)
    return (jnp.concatenate(vals_parts, axis=0),
            jnp.concatenate(idx_parts, axis=0))


# speedup vs baseline: 4.9009x; 1.0654x over previous
import functools

import jax
import jax.numpy as jnp
from jax import lax
from jax.experimental import pallas as pl
from jax.experimental.pallas import tpu as pltpu
from jax.experimental.pallas import tpu_sc as plsc

K_NN = 16
I32MAX = 2**31 - 1
LANES = 16
CHUNK = 128


def _dist_body(src_ref, dst_ref, dist_ref, cmins_ref):
    src = src_ref[...]
    dst = dst_ref[...]
    qb = src.shape[0]
    nb = dst.shape[0]
    nchb = nb // CHUNK
    s2 = jnp.sum(src * src, axis=-1, keepdims=True)
    d2 = jnp.sum(dst * dst, axis=-1)[None, :]
    ab = lax.dot_general(src, dst, (((1,), (1,)), ((), ())),
                         preferred_element_type=jnp.float32)
    dist2 = jnp.maximum(s2 + d2 - 2.0 * ab, 0.0)
    safe = jnp.where(dist2 > 0, dist2, 1.0)
    dist = jnp.where(dist2 > 0, jnp.sqrt(safe), 0.0)
    dist_ref[...] = dist
    cmins_ref[0] = jnp.min(dist.reshape(qb, nchb, CHUNK), axis=2)


def _thresh_body(cmins_ref, thresh_ref):
    w = cmins_ref[...]
    qb = w.shape[1]
    for _ in range(K_NN - 1):
        m = jnp.min(w, axis=(0, 2), keepdims=True)
        w = jnp.where(w == m, jnp.inf, w)
    t16 = jnp.min(w, axis=(0, 2))
    thresh_ref[...] = jnp.broadcast_to(t16.reshape(qb, 1), (qb, LANES))


def _tc_dist(src, dst):
    q, d = src.shape
    n, _ = dst.shape
    qb = min(256, q)
    nb = min(4096, n)
    nsteps = n // nb
    nch = n // CHUNK
    nchb = nb // CHUNK
    dist, cmins = pl.pallas_call(
        _dist_body,
        grid=(q // qb, nsteps),
        in_specs=[
            pl.BlockSpec((qb, d), lambda i, j: (i, 0)),
            pl.BlockSpec((nb, d), lambda i, j: (j, 0)),
        ],
        out_specs=[
            pl.BlockSpec((qb, nb), lambda i, j: (i, j)),
            pl.BlockSpec((1, qb, nchb), lambda i, j: (j, i, 0)),
        ],
        out_shape=[
            jax.ShapeDtypeStruct((q, n), jnp.float32),
            jax.ShapeDtypeStruct((nsteps, q, nchb), jnp.float32),
        ],
        compiler_params=pltpu.CompilerParams(
            dimension_semantics=("parallel", "arbitrary"),
        ),
    )(src, dst)
    thresh = pl.pallas_call(
        _thresh_body,
        grid=(q // 512,),
        in_specs=[pl.BlockSpec((nsteps, 512, nchb), lambda i: (0, i, 0))],
        out_specs=pl.BlockSpec((512, LANES), lambda i: (i, 0)),
        out_shape=jax.ShapeDtypeStruct((q, LANES), jnp.float32),
        compiler_params=pltpu.CompilerParams(
            dimension_semantics=("arbitrary",),
        ),
    )(cmins)
    return dist, cmins, thresh


def _gat(v, idx):
    return v.at[idx].get(mode="promise_in_bounds")


def _tree_max(v, lane):
    for sh in (8, 4, 2, 1):
        v = jnp.maximum(v, _gat(v, (lane + sh) & (LANES - 1)))
    return v


def _sc_topk_build(q, n):
    nch = n // CHUNK
    nsteps = n // min(4096, n)
    nchb = nch // nsteps
    nw = 32
    rpw = q // nw
    capl = n // LANES
    mesh = plsc.VectorSubcoreMesh(core_axis_name="c", subcore_axis_name="s")

    @functools.partial(
        pl.kernel,
        mesh=mesh,
        compiler_params=pltpu.CompilerParams(needs_layout_passes=False),
        out_type=[
            jax.ShapeDtypeStruct((q, K_NN), jnp.float32),
            jax.ShapeDtypeStruct((q, K_NN), jnp.int32),
        ],
        scratch_types=[
            pltpu.VMEM((n,), jnp.float32),
            pltpu.VMEM((n,), jnp.float32),
            pltpu.VMEM((nch,), jnp.float32),
            pltpu.VMEM((nch,), jnp.float32),
            pltpu.VMEM((rpw * LANES,), jnp.float32),
            pltpu.VMEM((LANES * capl,), jnp.float32),
            pltpu.VMEM((LANES * capl,), jnp.int32),
            pltpu.VMEM((nch,), jnp.int32),
            pltpu.VMEM((nch + LANES,), jnp.int32),
            pltpu.VMEM((rpw, K_NN), jnp.float32),
            pltpu.VMEM((rpw, K_NN), jnp.int32),
            pltpu.SemaphoreType.DMA,
            pltpu.SemaphoreType.DMA,
        ],
    )
    def sc_topk(dist_hbm, cmins_hbm, thresh_hbm, vals_hbm, idx_hbm,
                row_a, row_b, cm_a, cm_b, th_v, cand_v, cand_i, hl_v, hlc_v,
                out_v, out_i, sem_a, sem_b):
        wid = lax.axis_index("s") * 2 + lax.axis_index("c")
        row0 = wid * rpw
        lane = lax.broadcasted_iota(jnp.int32, (LANES,), 0)
        base = lane * capl
        inf_vec = jnp.full((LANES,), jnp.inf, jnp.float32)
        imax_vec = jnp.full((LANES,), I32MAX, jnp.int32)
        zero_i = jnp.zeros((LANES,), jnp.int32)

        pltpu.sync_copy(thresh_hbm.at[pl.ds(row0 * LANES, rpw * LANES)], th_v)

        def process(row_v, cm_v, r_local):
            toff = pl.multiple_of(r_local * LANES, LANES)
            th_vec = th_v[pl.ds(toff, LANES)]
            th = th_vec[0]

            caph = nch // LANES

            def hb(u, hcnt):
                cmv = cm_v[pl.ds(pl.multiple_of(u * LANES, LANES), LANES)]
                mask = cmv <= th_vec
                plsc.store_scatter(hl_v, [lane * caph + hcnt],
                                   lane + u * LANES, mask=mask)
                return hcnt + jnp.where(mask, jnp.int32(1), jnp.int32(0))
            hcnt = lax.fori_loop(0, caph, hb, zero_i)

            s = hcnt
            for sh in (1, 2, 4, 8):
                g = _gat(s, (lane - sh) & (LANES - 1))
                s = s + jnp.where(lane >= sh, g, 0)
            excl = s - hcnt
            nhit = s[LANES - 1]
            maxh = _tree_max(hcnt, lane)[0]

            def hcomp(k, carry):
                m = k < hcnt
                ids = plsc.load_gather(hl_v, [lane * caph + k])
                plsc.store_scatter(hlc_v, [excl + k], ids, mask=m)
                return carry
            lax.fori_loop(0, maxh, hcomp, jnp.int32(0))

            def pb2(it, cnt_l):
                hc = plsc.load_gather(hlc_v, [zero_i + it])[0]
                cb = pl.multiple_of(hc * CHUNK, CHUNK)
                for u in range(CHUNK // LANES):
                    v = row_v[pl.ds(cb + u * LANES, LANES)]
                    mask = v <= th_vec
                    plsc.store_scatter(cand_v, [base + cnt_l], v, mask=mask)
                    plsc.store_scatter(cand_i, [base + cnt_l],
                                       lane + (hc * CHUNK + u * LANES),
                                       mask=mask)
                    cnt_l = cnt_l + jnp.where(mask, jnp.int32(1),
                                              jnp.int32(0))
                return cnt_l

            cnt_l = lax.fori_loop(0, nhit, pb2, zero_i)
            maxc = _tree_max(cnt_l, lane)[0]

            def cstep(v, i, k, j):
                p = lane ^ j
                pv_ = _gat(v, p)
                pi_ = _gat(i, p)
                less = (pv_ < v) | ((pv_ == v) & (pi_ < i))
                grtr = (pv_ > v) | ((pv_ == v) & (pi_ > i))
                lo = (lane & j) == 0
                dir_up = (lane & k) == 0
                takemin = lo == dir_up
                sel = jnp.where(takemin, less, grtr)
                return jnp.where(sel, pv_, v), jnp.where(sel, pi_, i)

            def cmerge(j, rvri):
                rv, ri = rvri
                m = j < cnt_l
                v = plsc.load_gather(cand_v, [base + j])
                i = plsc.load_gather(cand_i, [base + j])
                v = jnp.where(m, v, jnp.inf)
                i = jnp.where(m, i, I32MAX)
                for k, jj in ((2, 1), (4, 2), (4, 1), (8, 4), (8, 2),
                              (8, 1), (16, 8), (16, 4), (16, 2), (16, 1)):
                    v, i = cstep(v, i, k, jj)
                wv = _gat(v, (LANES - 1) - lane)
                wi = _gat(i, (LANES - 1) - lane)
                keep = (rv < wv) | ((rv == wv) & (ri < wi))
                rv = jnp.where(keep, rv, wv)
                ri = jnp.where(keep, ri, wi)
                for jj in (8, 4, 2, 1):
                    rv, ri = cstep(rv, ri, 16, jj)
                return rv, ri

            rv, ri = lax.fori_loop(0, maxc, cmerge, (inf_vec, imax_vec))
            out_v[r_local] = rv
            out_i[r_local] = ri

        def fetch(r, row_buf, cm_buf, sem):
            rg = row0 + r
            pltpu.async_copy(dist_hbm.at[rg], row_buf, sem)
            for j in range(nsteps):
                pltpu.async_copy(
                    cmins_hbm.at[pl.ds(j * q * nchb + rg * nchb, nchb)],
                    cm_buf.at[pl.ds(j * nchb, nchb)], sem)

        def drain(row_buf, cm_buf, sem):
            pltpu.make_async_copy(dist_hbm.at[0], row_buf, sem).wait()
            pltpu.make_async_copy(
                cmins_hbm.at[pl.ds(0, nch)], cm_buf, sem).wait()

        fetch(0, row_a, cm_a, sem_a)

        def pair(rp, carry):
            r = 2 * rp
            drain(row_a, cm_a, sem_a)
            fetch(r + 1, row_b, cm_b, sem_b)
            process(row_a, cm_a, r)
            drain(row_b, cm_b, sem_b)
            fetch(jnp.minimum(r + 2, rpw - 1), row_a, cm_a, sem_a)
            process(row_b, cm_b, r + 1)
            return carry

        lax.fori_loop(0, rpw // 2, pair, jnp.int32(0))
        drain(row_a, cm_a, sem_a)

        pltpu.sync_copy(out_v, vals_hbm.at[pl.ds(row0, rpw)])
        pltpu.sync_copy(out_i, idx_hbm.at[pl.ds(row0, rpw)])

    return sc_topk


def kernel(src, dst):
    q, _ = src.shape
    n, _ = dst.shape
    groups = 4
    qg = q // groups
    sc_topk = _sc_topk_build(qg, n)
    vals_parts, idx_parts = [], []
    for g in range(groups):
        dist, cmins, thresh = _tc_dist(src[g * qg:(g + 1) * qg], dst)
        v, i = sc_topk(dist, cmins.reshape(-1), thresh.reshape(-1))
        vals_parts.append(v)
        idx_parts.append(i)
    return (jnp.concatenate(vals_parts, axis=0),
            jnp.concatenate(idx_parts, axis=0))
